# baseline (speedup 1.0000x reference)
import jax
import jax.numpy as jnp
from jax import lax
from jax.experimental import pallas as pl
from jax.experimental.pallas import tpu as pltpu
from jax.experimental.pallas import tpu_sc as plsc

N = 10000
NPAD = 10240
D = 256
H = 256
E_OUT = 4
NE = 160000

NC = 2
NS = 16
C = 128
NCH = 80
EPAD = NS * NCH * C
DW = 16
RB = 1000
GRID = N // RB

_sc_mesh = plsc.VectorSubcoreMesh(
    core_axis_name="c", subcore_axis_name="s", num_cores=NC, num_subcores=NS)


def _deg_body(pk, out, pkv, idxd, ones, zbuf, acc):
    c = lax.axis_index("c")
    s = lax.axis_index("s")
    pltpu.sync_copy(pk.at[s], pkv)

    def _fill(i, _):
        zbuf[i, :] = jnp.zeros((DW,), jnp.float32)
        ones[i, :] = jnp.ones((DW,), jnp.float32)
        return 0
    lax.fori_loop(0, C, _fill, 0)
    for t in range(5):
        pltpu.sync_copy(zbuf, acc.at[pl.ds(s * 640 + t * C, C)])
    plsc.subcore_barrier()

    def _chunk(j, _):
        for k in range(8):
            idxd[0, pl.ds(k * 16, 16)] = pkv[j, pl.ds(k * 16, 16)] & 32767
        pltpu.sync_copy(ones, acc.at[idxd.at[0]], add=True)
        return 0
    lax.fori_loop(0, NCH, _chunk, 0)
    plsc.subcore_barrier()

    @pl.when(c == 0)
    def _():
        pltpu.sync_copy(acc.at[pl.ds(s * 640, 640)],
                        out.at[pl.ds(s * 640, 640)])


_deg_kernel = pl.kernel(
    _deg_body,
    out_type=jax.ShapeDtypeStruct((NPAD, DW), jnp.float32),
    mesh=_sc_mesh,
    compiler_params=pltpu.CompilerParams(use_tc_tiling_on_sc=False),
    scratch_types=[
        pltpu.VMEM((NCH, C), jnp.int32),
        pltpu.VMEM((1, C), jnp.int32),
        pltpu.VMEM((C, DW), jnp.float32),
        pltpu.VMEM((C, DW), jnp.float32),
        pltpu.VMEM_SHARED((NPAD, DW), jnp.float32),
    ],
)


def _agg_body(h2d, pk, out, pkv, idxg, idxd, rowbuf, acc, sem):
    c = lax.axis_index("c")
    s = lax.axis_index("s")
    pltpu.sync_copy(pk.at[s], pkv)

    def _unpack(j, b):
        for k in range(8):
            p = pkv[j, pl.ds(k * 16, 16)]
            idxg[b, k // 4, pl.ds((k % 4) * 16, 16)] = (p >> 15) + c
            idxd[b, pl.ds(k * 16, 16)] = p & 32767

    def _gather(b):
        pltpu.async_copy(h2d.at[idxg.at[b, 0]],
                         rowbuf.at[b, pl.ds(0, 64)], sem.at[2 * b])
        pltpu.async_copy(h2d.at[idxg.at[b, 1]],
                         rowbuf.at[b, pl.ds(64, 64)], sem.at[2 * b + 1])

    def _gwait(b):
        pltpu.make_async_copy(h2d.at[idxg.at[b, 0]],
                              rowbuf.at[b, pl.ds(0, 64)], sem.at[2 * b]).wait()
        pltpu.make_async_copy(h2d.at[idxg.at[b, 1]],
                              rowbuf.at[b, pl.ds(64, 64)],
                              sem.at[2 * b + 1]).wait()

    def _zfill(i, _):
        for k in range(8):
            rowbuf[0, i, pl.ds(k * 16, 16)] = jnp.zeros((16,), jnp.float32)
        return 0
    lax.fori_loop(0, C, _zfill, 0)
    for t in range(5):
        pltpu.sync_copy(rowbuf.at[0], acc.at[pl.ds(s * 640 + t * C, C)])
    _unpack(0, 0)
    _gather(0)
    plsc.subcore_barrier()

    @pl.loop(0, NCH, step=2)
    def _pair(j):
        _unpack(j + 1, 1)
        _gather(1)
        _gwait(0)
        pltpu.sync_copy(rowbuf.at[0], acc.at[idxd.at[0]], add=True)

        @pl.when(j + 2 < NCH)
        def _():
            _unpack(j + 2, 0)
            _gather(0)

        _gwait(1)
        pltpu.sync_copy(rowbuf.at[1], acc.at[idxd.at[1]], add=True)

    plsc.subcore_barrier()
    pltpu.sync_copy(acc.at[pl.ds(s * 640, 640)],
                    out.at[c, pl.ds(s * 640, 640)])


_agg_kernel = pl.kernel(
    _agg_body,
    out_type=jax.ShapeDtypeStruct((NC, NPAD, 128), jnp.float32),
    mesh=_sc_mesh,
    compiler_params=pltpu.CompilerParams(use_tc_tiling_on_sc=False),
    scratch_types=[
        pltpu.VMEM((NCH, C), jnp.int32),
        pltpu.VMEM((2, 2, C // 2), jnp.int32),
        pltpu.VMEM((2, C), jnp.int32),
        pltpu.VMEM((2, C, 128), jnp.float32),
        pltpu.VMEM_SHARED((NPAD, 128), jnp.float32),
        pltpu.SemaphoreType.DMA((4,)),
    ],
)


def _tc1_body(x_ref, w1_ref, deg_ref, h1p_ref, dinv_ref):
    dinv = lax.rsqrt(deg_ref[:, 0:1] + 1.0)
    h = jnp.dot(x_ref[...], w1_ref[...], preferred_element_type=jnp.float32)
    hp = h * dinv
    h1p_ref[:, 0, :] = hp[:, :128]
    h1p_ref[:, 1, :] = hp[:, 128:]
    dinv_ref[...] = dinv


_tc1 = pl.pallas_call(
    _tc1_body,
    grid=(GRID,),
    in_specs=[
        pl.BlockSpec((RB, D), lambda i: (i, 0)),
        pl.BlockSpec((D, H), lambda i: (0, 0)),
        pl.BlockSpec((RB, DW), lambda i: (i, 0)),
    ],
    out_specs=[
        pl.BlockSpec((RB, 2, 128), lambda i: (i, 0, 0)),
        pl.BlockSpec((RB, 1), lambda i: (i, 0)),
    ],
    out_shape=[
        jax.ShapeDtypeStruct((N, 2, 128), jnp.float32),
        jax.ShapeDtypeStruct((N, 1), jnp.float32),
    ],
)


def _tc2_body(a0_ref, a1_ref, h1p_ref, dinv_ref, b1_ref, w2_ref, h2p_ref):
    dinv = dinv_ref[...]
    aggc = jnp.concatenate(
        [a0_ref[0], a1_ref[0]], axis=-1).astype(jnp.float32)
    h1p = jnp.concatenate(
        [h1p_ref[:, 0, :], h1p_ref[:, 1, :]], axis=-1).astype(jnp.float32)
    pre = (aggc + h1p) * dinv + b1_ref[...]
    r = jnp.maximum(pre, 0.0)
    h2 = jnp.dot(r, w2_ref[...], preferred_element_type=jnp.float32)
    h2p = h2 * dinv
    h2p_ref[:, 0, :] = h2p[:, :128]
    h2p_ref[:, 1, :] = h2p[:, 128:]


_AGG_SPECS = [
    pl.BlockSpec((1, RB, 128), lambda i: (0, i, 0)),
    pl.BlockSpec((1, RB, 128), lambda i: (1, i, 0)),
]

_tc2 = pl.pallas_call(
    _tc2_body,
    grid=(GRID,),
    in_specs=[
        *_AGG_SPECS,
        pl.BlockSpec((RB, 2, 128), lambda i: (i, 0, 0)),
        pl.BlockSpec((RB, 1), lambda i: (i, 0)),
        pl.BlockSpec((H,), lambda i: (0,)),
        pl.BlockSpec((H, H), lambda i: (0, 0)),
    ],
    out_specs=pl.BlockSpec((RB, 2, 128), lambda i: (i, 0, 0)),
    out_shape=jax.ShapeDtypeStruct((N, 2, 128), jnp.float32),
)


def _tc3_body(a0_ref, a1_ref, h2p_ref, dinv_ref, b2_ref,
              wc1_ref, bc1_ref, wc2_ref, bc2_ref, out_ref):
    dinv = dinv_ref[...]
    aggc = jnp.concatenate(
        [a0_ref[0], a1_ref[0]], axis=-1).astype(jnp.float32)
    h2p = jnp.concatenate(
        [h2p_ref[:, 0, :], h2p_ref[:, 1, :]], axis=-1).astype(jnp.float32)
    pre = (aggc + h2p) * dinv + b2_ref[...]
    t = jnp.tanh(pre)
    e = jnp.maximum(
        jnp.dot(t, wc1_ref[...], preferred_element_type=jnp.float32)
        + bc1_ref[...], 0.0)
    e2 = jnp.maximum(
        jnp.dot(e, wc2_ref[...], preferred_element_type=jnp.float32)
        + bc2_ref[...], 0.0)
    out_ref[...] = e2


_tc3 = pl.pallas_call(
    _tc3_body,
    grid=(GRID,),
    in_specs=[
        *_AGG_SPECS,
        pl.BlockSpec((RB, 2, 128), lambda i: (i, 0, 0)),
        pl.BlockSpec((RB, 1), lambda i: (i, 0)),
        pl.BlockSpec((H,), lambda i: (0,)),
        pl.BlockSpec((H, H), lambda i: (0, 0)),
        pl.BlockSpec((H,), lambda i: (0,)),
        pl.BlockSpec((H, E_OUT), lambda i: (0, 0)),
        pl.BlockSpec((E_OUT,), lambda i: (0,)),
    ],
    out_specs=pl.BlockSpec((RB, E_OUT), lambda i: (i, 0)),
    out_shape=jax.ShapeDtypeStruct((N, E_OUT), jnp.float32),
)


def kernel(x, edge_index, W1, b1, W2, b2, Wc1, bc1, Wc2, bc2):
    pk0 = (edge_index[0] << 16) | edge_index[1]
    fill = jnp.full((EPAD - NE,), NPAD - 1, jnp.int32)
    pk = jnp.concatenate([pk0, fill]).reshape(NS, NCH, C)

    deg = _deg_kernel(pk)
    h1p, dinv = _tc1(x, W1, deg)
    agg1 = _agg_kernel(h1p.reshape(2 * N, 128), pk)
    h2p = _tc2(agg1, agg1, h1p, dinv, b1, W2)
    agg2 = _agg_kernel(h2p.reshape(2 * N, 128), pk)
    out = _tc3(agg2, agg2, h2p, dinv, b2, Wc1, bc1, Wc2, bc2)
    return out


# speedup vs baseline: 1.0036x; 1.0036x over previous
import jax
import jax.numpy as jnp
from jax import lax
from jax.experimental import pallas as pl
from jax.experimental.pallas import tpu as pltpu
from jax.experimental.pallas import tpu_sc as plsc

N = 10000
NPAD = 10240
D = 256
H = 256
E_OUT = 4
NE = 160000

NC = 2
NS = 16
C = 128
NCH = 80
EPAD = NS * NCH * C
DW = 16
RB = 1000
GRID = N // RB

_sc_mesh = plsc.VectorSubcoreMesh(
    core_axis_name="c", subcore_axis_name="s", num_cores=NC, num_subcores=NS)


def _deg_body(pk, out, pkv, idxd, ones, zbuf, acc):
    c = lax.axis_index("c")
    s = lax.axis_index("s")
    pltpu.sync_copy(pk.at[s], pkv)

    def _fill(i, _):
        zbuf[i, :] = jnp.zeros((DW,), jnp.float32)
        ones[i, :] = jnp.ones((DW,), jnp.float32)
        return 0
    lax.fori_loop(0, C, _fill, 0)
    for t in range(5):
        pltpu.sync_copy(zbuf, acc.at[pl.ds(s * 640 + t * C, C)])
    plsc.subcore_barrier()

    def _chunk(j, _):
        for k in range(8):
            idxd[0, pl.ds(k * 16, 16)] = pkv[j, pl.ds(k * 16, 16)] & 32767
        pltpu.sync_copy(ones, acc.at[idxd.at[0]], add=True)
        return 0
    lax.fori_loop(0, NCH, _chunk, 0)
    plsc.subcore_barrier()

    @pl.when(c == 0)
    def _():
        pltpu.sync_copy(acc.at[pl.ds(s * 640, 640)],
                        out.at[pl.ds(s * 640, 640)])


_deg_kernel = pl.kernel(
    _deg_body,
    out_type=jax.ShapeDtypeStruct((NPAD, DW), jnp.float32),
    mesh=_sc_mesh,
    compiler_params=pltpu.CompilerParams(use_tc_tiling_on_sc=False),
    scratch_types=[
        pltpu.VMEM((NCH, C), jnp.int32),
        pltpu.VMEM((1, C), jnp.int32),
        pltpu.VMEM((C, DW), jnp.float32),
        pltpu.VMEM((C, DW), jnp.float32),
        pltpu.VMEM_SHARED((NPAD, DW), jnp.float32),
    ],
)


def _agg_body(h2d, pk, out, pkv, idxg, idxd, rowbuf, acc, sem):
    c = lax.axis_index("c")
    s = lax.axis_index("s")
    pltpu.sync_copy(pk.at[s], pkv)

    def _unpack(j, b):
        for k in range(8):
            p = pkv[j, pl.ds(k * 16, 16)]
            idxg[b, pl.ds(k * 16, 16)] = (p >> 15) + c
            idxd[b, pl.ds(k * 16, 16)] = p & 32767

    def _zfill(i, _):
        for k in range(8):
            rowbuf[0, i, pl.ds(k * 16, 16)] = jnp.zeros((16,), jnp.float32)
        return 0
    lax.fori_loop(0, C, _zfill, 0)
    for t in range(5):
        pltpu.sync_copy(rowbuf.at[0], acc.at[pl.ds(s * 640 + t * C, C)])
    _unpack(0, 0)
    pltpu.async_copy(h2d.at[idxg.at[0]], rowbuf.at[0], sem.at[0])
    plsc.subcore_barrier()

    @pl.loop(0, NCH, step=2)
    def _pair(j):
        _unpack(j + 1, 1)
        pltpu.async_copy(h2d.at[idxg.at[1]], rowbuf.at[1], sem.at[1])
        pltpu.make_async_copy(
            h2d.at[idxg.at[0]], rowbuf.at[0], sem.at[0]).wait()
        pltpu.sync_copy(rowbuf.at[0], acc.at[idxd.at[0]], add=True)

        @pl.when(j + 2 < NCH)
        def _():
            _unpack(j + 2, 0)
            pltpu.async_copy(h2d.at[idxg.at[0]], rowbuf.at[0], sem.at[0])

        pltpu.make_async_copy(
            h2d.at[idxg.at[1]], rowbuf.at[1], sem.at[1]).wait()
        pltpu.sync_copy(rowbuf.at[1], acc.at[idxd.at[1]], add=True)

    plsc.subcore_barrier()
    pltpu.sync_copy(acc.at[pl.ds(s * 640, 640)],
                    out.at[c, pl.ds(s * 640, 640)])


_agg_kernel = pl.kernel(
    _agg_body,
    out_type=jax.ShapeDtypeStruct((NC, NPAD, 128), jnp.float32),
    mesh=_sc_mesh,
    compiler_params=pltpu.CompilerParams(use_tc_tiling_on_sc=False),
    scratch_types=[
        pltpu.VMEM((NCH, C), jnp.int32),
        pltpu.VMEM((2, C), jnp.int32),
        pltpu.VMEM((2, C), jnp.int32),
        pltpu.VMEM((2, C, 128), jnp.float32),
        pltpu.VMEM_SHARED((NPAD, 128), jnp.float32),
        pltpu.SemaphoreType.DMA((2,)),
    ],
)


def _tc1_body(x_ref, w1_ref, deg_ref, h1p_ref, dinv_ref):
    dinv = lax.rsqrt(deg_ref[:, 0:1] + 1.0)
    h = jnp.dot(x_ref[...], w1_ref[...], preferred_element_type=jnp.float32)
    hp = h * dinv
    h1p_ref[:, 0, :] = hp[:, :128]
    h1p_ref[:, 1, :] = hp[:, 128:]
    dinv_ref[...] = dinv


_tc1 = pl.pallas_call(
    _tc1_body,
    grid=(GRID,),
    in_specs=[
        pl.BlockSpec((RB, D), lambda i: (i, 0)),
        pl.BlockSpec((D, H), lambda i: (0, 0)),
        pl.BlockSpec((RB, DW), lambda i: (i, 0)),
    ],
    out_specs=[
        pl.BlockSpec((RB, 2, 128), lambda i: (i, 0, 0)),
        pl.BlockSpec((RB, 1), lambda i: (i, 0)),
    ],
    out_shape=[
        jax.ShapeDtypeStruct((N, 2, 128), jnp.float32),
        jax.ShapeDtypeStruct((N, 1), jnp.float32),
    ],
)


def _tc2_body(a0_ref, a1_ref, h1p_ref, dinv_ref, b1_ref, w2_ref, h2p_ref):
    dinv = dinv_ref[...]
    aggc = jnp.concatenate(
        [a0_ref[0], a1_ref[0]], axis=-1).astype(jnp.float32)
    h1p = jnp.concatenate(
        [h1p_ref[:, 0, :], h1p_ref[:, 1, :]], axis=-1).astype(jnp.float32)
    pre = (aggc + h1p) * dinv + b1_ref[...]
    r = jnp.maximum(pre, 0.0)
    h2 = jnp.dot(r, w2_ref[...], preferred_element_type=jnp.float32)
    h2p = h2 * dinv
    h2p_ref[:, 0, :] = h2p[:, :128]
    h2p_ref[:, 1, :] = h2p[:, 128:]


_AGG_SPECS = [
    pl.BlockSpec((1, RB, 128), lambda i: (0, i, 0)),
    pl.BlockSpec((1, RB, 128), lambda i: (1, i, 0)),
]

_tc2 = pl.pallas_call(
    _tc2_body,
    grid=(GRID,),
    in_specs=[
        *_AGG_SPECS,
        pl.BlockSpec((RB, 2, 128), lambda i: (i, 0, 0)),
        pl.BlockSpec((RB, 1), lambda i: (i, 0)),
        pl.BlockSpec((H,), lambda i: (0,)),
        pl.BlockSpec((H, H), lambda i: (0, 0)),
    ],
    out_specs=pl.BlockSpec((RB, 2, 128), lambda i: (i, 0, 0)),
    out_shape=jax.ShapeDtypeStruct((N, 2, 128), jnp.float32),
)


def _tc3_body(a0_ref, a1_ref, h2p_ref, dinv_ref, b2_ref,
              wc1_ref, bc1_ref, wc2_ref, bc2_ref, out_ref):
    dinv = dinv_ref[...]
    aggc = jnp.concatenate(
        [a0_ref[0], a1_ref[0]], axis=-1).astype(jnp.float32)
    h2p = jnp.concatenate(
        [h2p_ref[:, 0, :], h2p_ref[:, 1, :]], axis=-1).astype(jnp.float32)
    pre = (aggc + h2p) * dinv + b2_ref[...]
    t = jnp.tanh(pre)
    e = jnp.maximum(
        jnp.dot(t, wc1_ref[...], preferred_element_type=jnp.float32)
        + bc1_ref[...], 0.0)
    e2 = jnp.maximum(
        jnp.dot(e, wc2_ref[...], preferred_element_type=jnp.float32)
        + bc2_ref[...], 0.0)
    out_ref[...] = e2


_tc3 = pl.pallas_call(
    _tc3_body,
    grid=(GRID,),
    in_specs=[
        *_AGG_SPECS,
        pl.BlockSpec((RB, 2, 128), lambda i: (i, 0, 0)),
        pl.BlockSpec((RB, 1), lambda i: (i, 0)),
        pl.BlockSpec((H,), lambda i: (0,)),
        pl.BlockSpec((H, H), lambda i: (0, 0)),
        pl.BlockSpec((H,), lambda i: (0,)),
        pl.BlockSpec((H, E_OUT), lambda i: (0, 0)),
        pl.BlockSpec((E_OUT,), lambda i: (0,)),
    ],
    out_specs=pl.BlockSpec((RB, E_OUT), lambda i: (i, 0)),
    out_shape=jax.ShapeDtypeStruct((N, E_OUT), jnp.float32),
)


def kernel(x, edge_index, W1, b1, W2, b2, Wc1, bc1, Wc2, bc2):
    pk0 = (edge_index[0] << 16) | edge_index[1]
    fill = jnp.full((EPAD - NE,), NPAD - 1, jnp.int32)
    pk = jnp.concatenate([pk0, fill]).reshape(NS, NCH, C)

    deg = _deg_kernel(pk)
    h1p, dinv = _tc1(x, W1, deg)
    agg1 = _agg_kernel(h1p.reshape(2 * N, 128), pk)
    h2p = _tc2(agg1, agg1, h1p, dinv, b1, W2)
    agg2 = _agg_kernel(h2p.reshape(2 * N, 128), pk)
    out = _tc3(agg2, agg2, h2p, dinv, b2, Wc1, bc1, Wc2, bc2)
    return out
